# Initial kernel scaffold; baseline (speedup 1.0000x reference)
import functools

import jax
import jax.numpy as jnp
from jax import lax
from jax.experimental import pallas as pl
from jax.experimental.pallas import tpu as pltpu
from jax.experimental.pallas import tpu_sc as plsc

NC = 2
NS = 16
LANES = 16
CHUNK = 128
GROUP = 16


def _sc_agg_builder(n, e_pad, dh, acc_rows, with_cnt):
  two_n = 2 * n
  rows_per_sub_w = two_n // NS
  rows_per_sub_z = acc_rows // NS
  n_groups = e_pad // (NS * GROUP * CHUNK)

  mesh = plsc.VectorSubcoreMesh(core_axis_name="c", subcore_axis_name="s")

  out_type = [jax.ShapeDtypeStruct((NC * two_n, dh), jnp.float32)]
  if with_cnt:
    out_type.append(jax.ShapeDtypeStruct((two_n, LANES), jnp.float32))

  scratch = [
      pltpu.VMEM((GROUP, CHUNK), jnp.int32),
      pltpu.VMEM((GROUP, CHUNK), jnp.int32),
      pltpu.VMEM((CHUNK, dh), jnp.float32),
      pltpu.VMEM((CHUNK, LANES), jnp.float32),
      pltpu.VMEM((CHUNK, LANES), jnp.float32),
      pltpu.VMEM((CHUNK, dh), jnp.float32),
      pltpu.VMEM_SHARED((acc_rows, dh), jnp.float32),
      pltpu.VMEM_SHARED((acc_rows, LANES), jnp.float32),
  ]

  def body(xa_hbm, xb_hbm, src_hbm, idx_hbm, *refs):
    if with_cnt:
      agg_hbm, cnt_hbm = refs[0], refs[1]
      refs = refs[2:]
    else:
      agg_hbm = refs[0]
      refs = refs[1:]
    src_v, idx_v, rows_v, ones_v, zeros16_v, zero_v, acc_sh, cnt_sh = refs

    cid = lax.axis_index("c")
    sid = lax.axis_index("s")

    @pl.loop(0, CHUNK)
    def _(i):
      @pl.loop(0, dh, step=LANES)
      def _(j):
        zero_v.at[pl.ds(i, 1), pl.ds(j, LANES)][...] = jnp.zeros(
            (1, LANES), jnp.float32)
      if with_cnt:
        ones_v.at[pl.ds(i, 1), pl.ds(0, LANES)][...] = jnp.ones(
            (1, LANES), jnp.float32)
        zeros16_v.at[pl.ds(i, 1), pl.ds(0, LANES)][...] = jnp.zeros(
            (1, LANES), jnp.float32)

    zbase = sid * rows_per_sub_z

    @pl.loop(0, rows_per_sub_z, step=CHUNK)
    def _(k):
      pltpu.sync_copy(zero_v, acc_sh.at[pl.ds(zbase + k, CHUNK)])
      if with_cnt:
        pltpu.sync_copy(zeros16_v, cnt_sh.at[pl.ds(zbase + k, CHUNK)])

    plsc.subcore_barrier()

    def process(table_hbm, do_cnt):
      row0 = sid * (n_groups * GROUP)

      @pl.loop(0, n_groups)
      def _(g):
        ro = row0 + g * GROUP
        pltpu.sync_copy(src_hbm.at[pl.ds(ro, GROUP)], src_v)
        pltpu.sync_copy(idx_hbm.at[pl.ds(ro, GROUP)], idx_v)

        @pl.loop(0, GROUP)
        def _(j):
          pltpu.sync_copy(table_hbm.at[src_v.at[j]], rows_v)
          pltpu.sync_copy(rows_v, acc_sh.at[idx_v.at[j]], add=True)
          if do_cnt:
            pltpu.sync_copy(ones_v, cnt_sh.at[idx_v.at[j]], add=True)

    @pl.when(cid == 0)
    def _():
      process(xa_hbm, with_cnt)

    @pl.when(cid == 1)
    def _():
      process(xb_hbm, False)

    plsc.subcore_barrier()

    wbase = sid * rows_per_sub_w
    pltpu.sync_copy(acc_sh.at[pl.ds(wbase, rows_per_sub_w)],
                    agg_hbm.at[pl.ds(cid * two_n + wbase, rows_per_sub_w)])
    if with_cnt:
      @pl.when(cid == 0)
      def _():
        pltpu.sync_copy(cnt_sh.at[pl.ds(wbase, rows_per_sub_w)],
                        cnt_hbm.at[pl.ds(wbase, rows_per_sub_w)])

  return pl.kernel(body, out_type=out_type, mesh=mesh, scratch_types=scratch)


def _tc_layer(x, agg, cnt, w_rel, w_root, b, wc=None, bc=None, blk=1000):
  n, d = x.shape
  r_count, _, h = w_rel.shape
  dh = d // 2
  grid = (n // blk,)
  out_dim = wc.shape[1] if wc is not None else h
  nb = n // blk

  in_specs = [pl.BlockSpec((blk, d), lambda i: (i, 0))]
  for c in range(NC):
    for r in range(r_count):
      in_specs.append(pl.BlockSpec(
          (blk, dh),
          functools.partial(lambda i, off: (off + i, 0), off=c * 2 * nb + r * nb)))
  for r in range(r_count):
    in_specs.append(pl.BlockSpec(
        (blk, LANES),
        functools.partial(lambda i, off: (off + i, 0), off=r * nb)))
  in_specs.append(pl.BlockSpec((r_count, d, h), lambda i: (0, 0, 0)))
  in_specs.append(pl.BlockSpec((d, h), lambda i: (0, 0)))
  in_specs.append(pl.BlockSpec((1, h), lambda i: (0, 0)))
  extra = []
  if wc is not None:
    in_specs.append(pl.BlockSpec((h, out_dim), lambda i: (0, 0)))
    in_specs.append(pl.BlockSpec((1, out_dim), lambda i: (0, 0)))
    extra = [wc, bc.reshape(1, -1)]

  def body(*refs):
    x_ref = refs[0]
    agg_refs = refs[1:1 + NC * r_count]
    cnt_refs = refs[1 + NC * r_count:1 + NC * r_count + r_count]
    base = 1 + NC * r_count + r_count
    w_rel_ref, w_root_ref, b_ref = refs[base], refs[base + 1], refs[base + 2]
    if wc is not None:
      wc_ref, bc_ref = refs[base + 3], refs[base + 4]
    o_ref = refs[-1]

    acc = jnp.dot(x_ref[...], w_root_ref[...],
                  preferred_element_type=jnp.float32,
                  precision=lax.Precision.HIGHEST) + b_ref[...]
    for r in range(r_count):
      a = jnp.concatenate(
          [agg_refs[r][...], agg_refs[r_count + r][...]], axis=1)
      inv = 1.0 / jnp.maximum(cnt_refs[r][:, 0:1], 1.0)
      acc = acc + jnp.dot(a * inv, w_rel_ref[r],
                          preferred_element_type=jnp.float32,
                          precision=lax.Precision.HIGHEST)
    acc = jnp.maximum(acc, 0.0)
    if wc is not None:
      acc = jnp.dot(acc, wc_ref[...],
                    preferred_element_type=jnp.float32,
                    precision=lax.Precision.HIGHEST) + bc_ref[...]
    o_ref[...] = acc

  args = [x]
  for _ in range(NC * r_count):
    args.append(agg)
  for _ in range(r_count):
    args.append(cnt)
  args += [w_rel, w_root, b.reshape(1, -1)] + extra

  return pl.pallas_call(
      body,
      grid=grid,
      in_specs=in_specs,
      out_specs=pl.BlockSpec((blk, out_dim), lambda i: (i, 0)),
      out_shape=jax.ShapeDtypeStruct((n, out_dim), jnp.float32),
  )(*args)


def kernel(x, edge_index, edge_attr, W_rel1, W_root1, b1, W_rel2, W_root2, b2,
           Wc, bc):
  n, d = x.shape
  e = edge_index.shape[1]
  dh = d // 2

  edges_per_pass = NS * GROUP * CHUNK
  e_pad = ((e + edges_per_pass - 1) // edges_per_pass) * edges_per_pass
  acc_rows = ((2 * n + 1 + NS * CHUNK - 1) // (NS * CHUNK)) * (NS * CHUNK)

  src = edge_index[0]
  dst = edge_index[1]
  rel = edge_attr[:, -1].astype(jnp.int32)
  idx = rel * n + dst
  pad = e_pad - e
  src_p = jnp.concatenate([src, jnp.zeros((pad,), jnp.int32)])
  idx_p = jnp.concatenate([idx, jnp.full((pad,), 2 * n, jnp.int32)])
  src2 = src_p.reshape(-1, CHUNK)
  idx2 = idx_p.reshape(-1, CHUNK)

  agg_fn1 = _sc_agg_builder(n, e_pad, dh, acc_rows, with_cnt=True)
  agg_fn2 = _sc_agg_builder(n, e_pad, dh, acc_rows, with_cnt=False)

  agg1, cnt = agg_fn1(x[:, :dh], x[:, dh:], src2, idx2)
  h = _tc_layer(x, agg1, cnt, W_rel1, W_root1, b1)
  agg2 = agg_fn2(h[:, :dh], h[:, dh:], src2, idx2)
  if isinstance(agg2, (list, tuple)):
    agg2 = agg2[0]
  return _tc_layer(h, agg2, cnt, W_rel2, W_root2, b2, Wc, bc)


# speedup vs baseline: 4.7231x; 4.7231x over previous
import functools

import jax
import jax.numpy as jnp
from jax import lax
from jax.experimental import pallas as pl
from jax.experimental.pallas import tpu as pltpu
from jax.experimental.pallas import tpu_sc as plsc

NC = 2
NS = 16
LANES = 16
CHUNK = 128
GROUP = 16


def _sc_agg_builder(npad, e_pad, dh, with_cnt):
  two_n = 2 * npad
  rows_per_sub = two_n // NS
  assert rows_per_sub % CHUNK == 0 and rows_per_sub % 8 == 0
  n_groups = e_pad // (NS * GROUP * CHUNK)
  assert e_pad % (NS * GROUP * CHUNK) == 0

  mesh = plsc.VectorSubcoreMesh(core_axis_name="c", subcore_axis_name="s",
                                num_cores=NC, num_subcores=NS)

  out_type = [jax.ShapeDtypeStruct((NC * two_n, dh), jnp.float32)]
  if with_cnt:
    out_type.append(jax.ShapeDtypeStruct((two_n, LANES), jnp.float32))

  scratch = [
      pltpu.VMEM((GROUP, CHUNK), jnp.int32),
      pltpu.VMEM((GROUP, CHUNK), jnp.int32),
      pltpu.VMEM((CHUNK, dh), jnp.float32),
      pltpu.VMEM((CHUNK, LANES), jnp.float32),
      pltpu.VMEM((CHUNK, dh), jnp.float32),
      pltpu.VMEM_SHARED((two_n, dh), jnp.float32),
      pltpu.VMEM_SHARED((two_n, LANES), jnp.float32),
  ]

  def body(xa_hbm, xb_hbm, src_hbm, idx_hbm, *refs):
    if with_cnt:
      agg_hbm, cnt_hbm = refs[0], refs[1]
      refs = refs[2:]
    else:
      agg_hbm = refs[0]
      refs = refs[1:]
    src_v, idx_v, rows_v, ones_v, zero_v, acc_sh, cnt_sh = refs

    cid = lax.axis_index("c")
    sid = lax.axis_index("s")

    @pl.loop(0, CHUNK)
    def _(i):
      @pl.loop(0, dh, step=LANES)
      def _(j):
        zero_v.at[pl.ds(i, 1), pl.ds(j, LANES)][...] = jnp.zeros(
            (1, LANES), jnp.float32)
      if with_cnt:
        ones_v.at[pl.ds(i, 1), pl.ds(0, LANES)][...] = jnp.ones(
            (1, LANES), jnp.float32)

    zbase = sid * rows_per_sub

    @pl.loop(0, rows_per_sub, step=CHUNK)
    def _(k):
      pltpu.sync_copy(zero_v, acc_sh.at[pl.ds(zbase + k, CHUNK)])
      if with_cnt:
        pltpu.sync_copy(zero_v.at[:, pl.ds(0, LANES)],
                        cnt_sh.at[pl.ds(zbase + k, CHUNK)])

    plsc.subcore_barrier()

    def process(table_hbm, do_cnt):
      row0 = sid * (n_groups * GROUP)

      @pl.loop(0, n_groups)
      def _(g):
        ro = row0 + g * GROUP
        pltpu.sync_copy(src_hbm.at[pl.ds(ro, GROUP)], src_v)
        pltpu.sync_copy(idx_hbm.at[pl.ds(ro, GROUP)], idx_v)

        @pl.loop(0, GROUP)
        def _(j):
          pltpu.sync_copy(table_hbm.at[src_v.at[j]], rows_v)
          pltpu.sync_copy(rows_v, acc_sh.at[idx_v.at[j]], add=True)
          if do_cnt:
            pltpu.sync_copy(ones_v, cnt_sh.at[idx_v.at[j]], add=True)

    @pl.when(cid == 0)
    def _():
      process(xa_hbm, with_cnt)

    @pl.when(cid == 1)
    def _():
      process(xb_hbm, False)

    plsc.subcore_barrier()

    wbase = sid * rows_per_sub
    pltpu.sync_copy(acc_sh.at[pl.ds(wbase, rows_per_sub)],
                    agg_hbm.at[pl.ds(cid * two_n + wbase, rows_per_sub)])
    if with_cnt:
      @pl.when(cid == 0)
      def _():
        pltpu.sync_copy(cnt_sh.at[pl.ds(wbase, rows_per_sub)],
                        cnt_hbm.at[pl.ds(wbase, rows_per_sub)])

  return pl.kernel(body, out_type=out_type, mesh=mesh, scratch_types=scratch,
                   compiler_params=pltpu.CompilerParams(
                       use_tc_tiling_on_sc=False))


def _tc_layer(x, agg, cnt, w_rel, w_root, b, wc=None, bc=None, blk=1000):
  n, d = x.shape
  r_count, _, h = w_rel.shape
  dh = d // 2
  grid = (n // blk,)
  out_dim = wc.shape[1] if wc is not None else h

  in_specs = [pl.BlockSpec((blk, d), lambda i: (i, 0))]
  for c in range(NC):
    for r in range(r_count):
      in_specs.append(pl.BlockSpec(
          (1, blk, dh),
          functools.partial(lambda i, k: (k, i, 0), k=c * r_count + r)))
  for r in range(r_count):
    in_specs.append(pl.BlockSpec(
        (1, blk, LANES),
        functools.partial(lambda i, k: (k, i, 0), k=r)))
  in_specs.append(pl.BlockSpec((r_count, d, h), lambda i: (0, 0, 0)))
  in_specs.append(pl.BlockSpec((d, h), lambda i: (0, 0)))
  in_specs.append(pl.BlockSpec((1, h), lambda i: (0, 0)))
  extra = []
  if wc is not None:
    in_specs.append(pl.BlockSpec((h, out_dim), lambda i: (0, 0)))
    in_specs.append(pl.BlockSpec((1, out_dim), lambda i: (0, 0)))
    extra = [wc, bc.reshape(1, -1)]

  def body(*refs):
    x_ref = refs[0]
    agg_refs = refs[1:1 + NC * r_count]
    cnt_refs = refs[1 + NC * r_count:1 + NC * r_count + r_count]
    base = 1 + NC * r_count + r_count
    w_rel_ref, w_root_ref, b_ref = refs[base], refs[base + 1], refs[base + 2]
    if wc is not None:
      wc_ref, bc_ref = refs[base + 3], refs[base + 4]
    o_ref = refs[-1]

    acc = jnp.dot(x_ref[...], w_root_ref[...],
                  preferred_element_type=jnp.float32,
                  precision=lax.Precision.HIGHEST) + b_ref[...]
    for r in range(r_count):
      a = jnp.concatenate(
          [agg_refs[r][0], agg_refs[r_count + r][0]], axis=1)
      inv = 1.0 / jnp.maximum(cnt_refs[r][0, :, 0:1], 1.0)
      acc = acc + jnp.dot(a * inv, w_rel_ref[r],
                          preferred_element_type=jnp.float32,
                          precision=lax.Precision.HIGHEST)
    acc = jnp.maximum(acc, 0.0)
    if wc is not None:
      acc = jnp.dot(acc, wc_ref[...],
                    preferred_element_type=jnp.float32,
                    precision=lax.Precision.HIGHEST) + bc_ref[...]
    o_ref[...] = acc

  args = [x]
  for _ in range(NC * r_count):
    args.append(agg)
  for _ in range(r_count):
    args.append(cnt)
  args += [w_rel, w_root, b.reshape(1, -1)] + extra

  return pl.pallas_call(
      body,
      grid=grid,
      in_specs=in_specs,
      out_specs=pl.BlockSpec((blk, out_dim), lambda i: (i, 0)),
      out_shape=jax.ShapeDtypeStruct((n, out_dim), jnp.float32),
  )(*args)


def kernel(x, edge_index, edge_attr, W_rel1, W_root1, b1, W_rel2, W_root2, b2,
           Wc, bc):
  n, d = x.shape
  e = edge_index.shape[1]
  dh = d // 2
  r_count = W_rel1.shape[0]

  npad = ((n + 1 + NS * CHUNK - 1) // (NS * CHUNK)) * (NS * CHUNK)
  edges_per_pass = NS * GROUP * CHUNK
  e_pad = ((e + edges_per_pass - 1) // edges_per_pass) * edges_per_pass

  src = edge_index[0]
  dst = edge_index[1]
  rel = edge_attr[:, -1].astype(jnp.int32)
  idx = rel * npad + dst
  pad = e_pad - e
  src_p = jnp.concatenate([src, jnp.zeros((pad,), jnp.int32)])
  idx_p = jnp.concatenate([idx, jnp.full((pad,), n, jnp.int32)])
  src2 = src_p.reshape(-1, CHUNK)
  idx2 = idx_p.reshape(-1, CHUNK)

  agg_fn1 = _sc_agg_builder(npad, e_pad, dh, with_cnt=True)
  agg_fn2 = _sc_agg_builder(npad, e_pad, dh, with_cnt=False)

  def trim_agg(a):
    return a.reshape(NC * r_count, npad, dh)[:, :n, :]

  agg1, cnt = agg_fn1(x[:, :dh], x[:, dh:], src2, idx2)
  cnt = cnt.reshape(r_count, npad, LANES)[:, :n, :]
  h = _tc_layer(x, trim_agg(agg1), cnt, W_rel1, W_root1, b1)
  agg2 = agg_fn2(h[:, :dh], h[:, dh:], src2, idx2)
  if isinstance(agg2, (list, tuple)):
    agg2 = agg2[0]
  return _tc_layer(h, trim_agg(agg2), cnt, W_rel2, W_root2, b2, Wc, bc)


# speedup vs baseline: 6.0111x; 1.2727x over previous
import functools

import jax
import jax.numpy as jnp
from jax import lax
from jax.experimental import pallas as pl
from jax.experimental.pallas import tpu as pltpu
from jax.experimental.pallas import tpu_sc as plsc

NC = 2
NS = 16
LANES = 16
CHUNK = 128
WAVE = 2
PAIR = 2 * WAVE

_SC_PARAMS = pltpu.CompilerParams(use_tc_tiling_on_sc=False)


def _mesh():
  return plsc.VectorSubcoreMesh(core_axis_name="c", subcore_axis_name="s",
                                num_cores=NC, num_subcores=NS)


def _zero_fill(buf, width):
  @pl.loop(0, CHUNK)
  def _(i):
    @pl.loop(0, width, step=LANES)
    def _(j):
      buf.at[pl.ds(i, 1), pl.ds(j, LANES)][...] = jnp.zeros(
          (1, LANES), jnp.float32)


def _sc_agg_builder(npad, e_pad, dh):
  two_n = 2 * npad
  rows_per_sub = two_n // NS
  assert rows_per_sub % CHUNK == 0 and rows_per_sub % 8 == 0
  cps = e_pad // (NS * CHUNK)
  n_pairs = cps // PAIR
  assert cps % PAIR == 0 and n_pairs >= 4 and n_pairs % 2 == 0

  out_type = jax.ShapeDtypeStruct((NC * two_n, dh), jnp.float32)

  scratch = [
      pltpu.VMEM((2, PAIR, CHUNK), jnp.int32),
      pltpu.VMEM((2, PAIR, CHUNK), jnp.int32),
      pltpu.VMEM((2, WAVE, CHUNK, dh), jnp.float32),
      pltpu.VMEM((CHUNK, dh), jnp.float32),
      pltpu.VMEM_SHARED((two_n, dh), jnp.float32),
      pltpu.SemaphoreType.DMA,
      pltpu.SemaphoreType.DMA,
      pltpu.SemaphoreType.DMA,
      pltpu.SemaphoreType.DMA,
  ]

  def body(xa_hbm, xb_hbm, src_hbm, idx_hbm, agg_hbm, *refs):
    src_v, idx_v, rows_v, zero_v, acc_sh, sem_g0, sem_g1, sem_i0, sem_i1 = refs
    gsems = (sem_g0, sem_g1)
    isems = (sem_i0, sem_i1)

    cid = lax.axis_index("c")
    sid = lax.axis_index("s")

    _zero_fill(zero_v, dh)
    zbase = sid * rows_per_sub

    @pl.loop(0, rows_per_sub, step=CHUNK)
    def _(k):
      pltpu.sync_copy(zero_v, acc_sh.at[pl.ds(zbase + k, CHUNK)])

    plsc.subcore_barrier()

    def process(table_hbm):
      row0 = sid * cps

      def idx_load(q, slot):
        ro = row0 + q * PAIR
        pltpu.async_copy(src_hbm.at[pl.ds(ro, PAIR)], src_v.at[slot],
                         isems[slot])
        pltpu.async_copy(idx_hbm.at[pl.ds(ro, PAIR)], idx_v.at[slot],
                         isems[slot])

      def idx_wait(slot):
        pltpu.make_async_copy(src_hbm.at[pl.ds(0, PAIR)], src_v.at[slot],
                              isems[slot]).wait()
        pltpu.make_async_copy(idx_hbm.at[pl.ds(0, PAIR)], idx_v.at[slot],
                              isems[slot]).wait()

      def fire_wave(islot, half, rslot):
        for j in range(WAVE):
          pltpu.async_copy(table_hbm.at[src_v.at[islot, half * WAVE + j]],
                           rows_v.at[rslot, j], gsems[rslot])

      def scatter_wave(islot, half, rslot):
        for j in range(WAVE):
          pltpu.make_async_copy(table_hbm.at[pl.ds(0, CHUNK)],
                                rows_v.at[rslot, j], gsems[rslot]).wait()
        for j in range(WAVE):
          pltpu.sync_copy(rows_v.at[rslot, j],
                          acc_sh.at[idx_v.at[islot, half * WAVE + j]],
                          add=True)

      def do_pair(q, islot, has_next):
        if has_next:
          idx_load(q + 1, 1 - islot)
        scatter_wave(islot, 0, 0)
        if has_next:
          idx_wait(1 - islot)
          fire_wave(1 - islot, 0, 0)
        scatter_wave(islot, 1, 1)
        if has_next:
          fire_wave(1 - islot, 1, 1)

      idx_load(0, 0)
      idx_wait(0)
      fire_wave(0, 0, 0)
      fire_wave(0, 1, 1)

      @pl.loop(0, n_pairs - 2, step=2)
      def _(p):
        do_pair(p, 0, True)
        do_pair(p + 1, 1, True)

      do_pair(n_pairs - 2, 0, True)
      do_pair(n_pairs - 1, 1, False)

    @pl.when(cid == 0)
    def _():
      process(xa_hbm)

    @pl.when(cid == 1)
    def _():
      process(xb_hbm)

    plsc.subcore_barrier()

    wbase = sid * rows_per_sub
    pltpu.sync_copy(acc_sh.at[pl.ds(wbase, rows_per_sub)],
                    agg_hbm.at[pl.ds(cid * two_n + wbase, rows_per_sub)])

  return pl.kernel(body, out_type=out_type, mesh=_mesh(),
                   scratch_types=scratch, compiler_params=_SC_PARAMS)


def _sc_cnt_builder(npad, e_pad):
  two_n = 2 * npad
  rows_per_sub = two_n // NS
  assert rows_per_sub % CHUNK == 0 and rows_per_sub % 8 == 0
  cps = e_pad // (NC * NS * CHUNK)
  n_blocks = cps // PAIR
  assert cps % PAIR == 0 and n_blocks >= 4 and n_blocks % 2 == 0

  out_type = jax.ShapeDtypeStruct((NC * two_n, LANES), jnp.float32)

  scratch = [
      pltpu.VMEM((2, PAIR, CHUNK), jnp.int32),
      pltpu.VMEM((CHUNK, LANES), jnp.float32),
      pltpu.VMEM((CHUNK, LANES), jnp.float32),
      pltpu.VMEM_SHARED((two_n, LANES), jnp.float32),
      pltpu.SemaphoreType.DMA,
      pltpu.SemaphoreType.DMA,
      pltpu.SemaphoreType.DMA,
  ]

  def body(idx_hbm, cnt_hbm, *refs):
    idx_v, ones_v, zero_v, cnt_sh, sem_i0, sem_i1, sem_c = refs
    isems = (sem_i0, sem_i1)

    cid = lax.axis_index("c")
    sid = lax.axis_index("s")

    _zero_fill(zero_v, LANES)

    @pl.loop(0, CHUNK)
    def _(i):
      ones_v.at[pl.ds(i, 1), pl.ds(0, LANES)][...] = jnp.ones(
          (1, LANES), jnp.float32)

    zbase = sid * rows_per_sub

    @pl.loop(0, rows_per_sub, step=CHUNK)
    def _(k):
      pltpu.sync_copy(zero_v, cnt_sh.at[pl.ds(zbase + k, CHUNK)])

    plsc.subcore_barrier()

    row0 = (cid * NS + sid) * cps

    def idx_load(q, slot):
      pltpu.async_copy(idx_hbm.at[pl.ds(row0 + q * PAIR, PAIR)],
                       idx_v.at[slot], isems[slot])

    def idx_wait(slot):
      pltpu.make_async_copy(idx_hbm.at[pl.ds(0, PAIR)], idx_v.at[slot],
                            isems[slot]).wait()

    def count_block(slot):
      for j in range(PAIR):
        pltpu.sync_copy(ones_v, cnt_sh.at[idx_v.at[slot, j]], add=True)

    def drain_block():
      pass

    idx_load(0, 0)
    idx_wait(0)

    @pl.loop(0, n_blocks - 2, step=2)
    def _(q):
      idx_load(q + 1, 1)
      count_block(0)
      idx_wait(1)
      drain_block()
      idx_load(q + 2, 0)
      count_block(1)
      idx_wait(0)
      drain_block()

    idx_load(n_blocks - 1, 1)
    count_block(0)
    idx_wait(1)
    drain_block()
    count_block(1)
    drain_block()

    plsc.subcore_barrier()

    pltpu.sync_copy(cnt_sh.at[pl.ds(zbase, rows_per_sub)],
                    cnt_hbm.at[pl.ds(cid * two_n + zbase, rows_per_sub)])

  return pl.kernel(body, out_type=out_type, mesh=_mesh(),
                   scratch_types=scratch, compiler_params=_SC_PARAMS)


def _tc_layer(x, agg, cnt, w_rel, w_root, b, wc=None, bc=None, blk=1000):
  n, d = x.shape
  r_count, _, h = w_rel.shape
  dh = d // 2
  grid = (n // blk,)
  out_dim = wc.shape[1] if wc is not None else h

  in_specs = [pl.BlockSpec((blk, d), lambda i: (i, 0))]
  for c in range(NC):
    for r in range(r_count):
      in_specs.append(pl.BlockSpec(
          (1, blk, dh),
          functools.partial(lambda i, k: (k, i, 0), k=c * r_count + r)))
  for c in range(NC):
    for r in range(r_count):
      in_specs.append(pl.BlockSpec(
          (1, 1, blk, LANES),
          functools.partial(lambda i, c, r: (c, r, i, 0), c=c, r=r)))
  in_specs.append(pl.BlockSpec((r_count, d, h), lambda i: (0, 0, 0)))
  in_specs.append(pl.BlockSpec((d, h), lambda i: (0, 0)))
  in_specs.append(pl.BlockSpec((1, h), lambda i: (0, 0)))
  extra = []
  if wc is not None:
    in_specs.append(pl.BlockSpec((h, out_dim), lambda i: (0, 0)))
    in_specs.append(pl.BlockSpec((1, out_dim), lambda i: (0, 0)))
    extra = [wc, bc.reshape(1, -1)]

  def body(*refs):
    x_ref = refs[0]
    agg_refs = refs[1:1 + NC * r_count]
    cnt_refs = refs[1 + NC * r_count:1 + 2 * NC * r_count]
    base = 1 + 2 * NC * r_count
    w_rel_ref, w_root_ref, b_ref = refs[base], refs[base + 1], refs[base + 2]
    if wc is not None:
      wc_ref, bc_ref = refs[base + 3], refs[base + 4]
    o_ref = refs[-1]

    acc = jnp.dot(x_ref[...], w_root_ref[...],
                  preferred_element_type=jnp.float32,
                  precision=lax.Precision.HIGHEST) + b_ref[...]
    for r in range(r_count):
      a = jnp.concatenate(
          [agg_refs[r][0], agg_refs[r_count + r][0]], axis=1)
      total = cnt_refs[r][0, 0, :, 0:1] + cnt_refs[r_count + r][0, 0, :, 0:1]
      inv = 1.0 / jnp.maximum(total, 1.0)
      acc = acc + jnp.dot(a * inv, w_rel_ref[r],
                          preferred_element_type=jnp.float32,
                          precision=lax.Precision.HIGHEST)
    acc = jnp.maximum(acc, 0.0)
    if wc is not None:
      acc = jnp.dot(acc, wc_ref[...],
                    preferred_element_type=jnp.float32,
                    precision=lax.Precision.HIGHEST) + bc_ref[...]
    o_ref[...] = acc

  args = [x]
  for _ in range(NC * r_count):
    args.append(agg)
  for _ in range(NC * r_count):
    args.append(cnt)
  args += [w_rel, w_root, b.reshape(1, -1)] + extra

  return pl.pallas_call(
      body,
      grid=grid,
      in_specs=in_specs,
      out_specs=pl.BlockSpec((blk, out_dim), lambda i: (i, 0)),
      out_shape=jax.ShapeDtypeStruct((n, out_dim), jnp.float32),
  )(*args)


def kernel(x, edge_index, edge_attr, W_rel1, W_root1, b1, W_rel2, W_root2, b2,
           Wc, bc):
  n, d = x.shape
  e = edge_index.shape[1]
  dh = d // 2
  r_count = W_rel1.shape[0]

  npad = ((n + 1 + NS * CHUNK - 1) // (NS * CHUNK)) * (NS * CHUNK)
  edges_per_pass = NC * NS * 2 * PAIR * CHUNK
  e_pad = ((e + edges_per_pass - 1) // edges_per_pass) * edges_per_pass

  src = edge_index[0]
  dst = edge_index[1]
  rel = edge_attr[:, -1].astype(jnp.int32)
  idx = rel * npad + dst
  pad = e_pad - e
  src_p = jnp.concatenate([src, jnp.zeros((pad,), jnp.int32)])
  idx_p = jnp.concatenate([idx, jnp.full((pad,), n, jnp.int32)])
  src2 = src_p.reshape(-1, CHUNK)
  idx2 = idx_p.reshape(-1, CHUNK)

  agg_fn = _sc_agg_builder(npad, e_pad, dh)
  cnt_fn = _sc_cnt_builder(npad, e_pad)

  def trim_agg(a):
    return a.reshape(NC * r_count, npad, dh)[:, :n, :]

  cnt = cnt_fn(idx2)
  cnt = cnt.reshape(NC, r_count, npad, LANES)[:, :, :n, :]
  agg1 = agg_fn(x[:, :dh], x[:, dh:], src2, idx2)
  h = _tc_layer(x, trim_agg(agg1), cnt, W_rel1, W_root1, b1)
  agg2 = agg_fn(h[:, :dh], h[:, dh:], src2, idx2)
  return _tc_layer(h, trim_agg(agg2), cnt, W_rel2, W_root2, b2, Wc, bc)


# speedup vs baseline: 7.7153x; 1.2835x over previous
import functools

import jax
import jax.numpy as jnp
from jax import lax
from jax.experimental import pallas as pl
from jax.experimental.pallas import tpu as pltpu
from jax.experimental.pallas import tpu_sc as plsc

NC = 2
NS = 16
LANES = 16
CHUNK = 128
WAVE = 2
PAIR = 2 * WAVE

_SC_PARAMS = pltpu.CompilerParams(use_tc_tiling_on_sc=False)


def _mesh():
  return plsc.VectorSubcoreMesh(core_axis_name="c", subcore_axis_name="s",
                                num_cores=NC, num_subcores=NS)


def _zero_fill(buf, width):
  @pl.loop(0, CHUNK)
  def _(i):
    @pl.loop(0, width, step=LANES)
    def _(j):
      buf.at[pl.ds(i, 1), pl.ds(j, LANES)][...] = jnp.zeros(
          (1, LANES), jnp.float32)


def _sc_agg_builder(npad, e_pad, dh):
  two_n = 2 * npad
  rows_per_sub = two_n // NS
  assert rows_per_sub % CHUNK == 0 and rows_per_sub % 8 == 0
  cps = e_pad // (NS * CHUNK)
  n_pairs = cps // PAIR
  assert cps % PAIR == 0 and n_pairs >= 4 and n_pairs % 2 == 0

  out_type = jax.ShapeDtypeStruct((NC * two_n, dh), jnp.float32)

  scratch = [
      pltpu.VMEM((2, PAIR, CHUNK), jnp.int32),
      pltpu.VMEM((2, PAIR, CHUNK), jnp.int32),
      pltpu.VMEM((2, WAVE, CHUNK, dh), jnp.float32),
      pltpu.VMEM((CHUNK, dh), jnp.float32),
      pltpu.VMEM_SHARED((two_n, dh), jnp.float32),
      pltpu.SemaphoreType.DMA,
      pltpu.SemaphoreType.DMA,
      pltpu.SemaphoreType.DMA,
      pltpu.SemaphoreType.DMA,
  ]

  def body(xa_hbm, xb_hbm, src_hbm, idx_hbm, agg_hbm, *refs):
    src_v, idx_v, rows_v, zero_v, acc_sh, sem_g0, sem_g1, sem_i0, sem_i1 = refs
    gsems = (sem_g0, sem_g1)
    isems = (sem_i0, sem_i1)

    cid = lax.axis_index("c")
    sid = lax.axis_index("s")

    _zero_fill(zero_v, dh)
    zbase = sid * rows_per_sub

    @pl.loop(0, rows_per_sub, step=CHUNK)
    def _(k):
      pltpu.sync_copy(zero_v, acc_sh.at[pl.ds(zbase + k, CHUNK)])

    plsc.subcore_barrier()

    def process(table_hbm):
      row0 = sid * cps

      def idx_load(q, slot):
        ro = row0 + q * PAIR
        pltpu.async_copy(src_hbm.at[pl.ds(ro, PAIR)], src_v.at[slot],
                         isems[slot])
        pltpu.async_copy(idx_hbm.at[pl.ds(ro, PAIR)], idx_v.at[slot],
                         isems[slot])

      def idx_wait(slot):
        pltpu.make_async_copy(src_hbm.at[pl.ds(0, PAIR)], src_v.at[slot],
                              isems[slot]).wait()
        pltpu.make_async_copy(idx_hbm.at[pl.ds(0, PAIR)], idx_v.at[slot],
                              isems[slot]).wait()

      def fire_wave(islot, half, rslot):
        for j in range(WAVE):
          pltpu.async_copy(table_hbm.at[src_v.at[islot, half * WAVE + j]],
                           rows_v.at[rslot, j], gsems[rslot])

      def scatter_wave(islot, half, rslot):
        for j in range(WAVE):
          pltpu.make_async_copy(table_hbm.at[pl.ds(0, CHUNK)],
                                rows_v.at[rslot, j], gsems[rslot]).wait()
        for j in range(WAVE):
          pltpu.sync_copy(rows_v.at[rslot, j],
                          acc_sh.at[idx_v.at[islot, half * WAVE + j]],
                          add=True)

      def do_pair(q, islot, has_next):
        if has_next:
          idx_load(q + 1, 1 - islot)
        scatter_wave(islot, 0, 0)
        if has_next:
          idx_wait(1 - islot)
          fire_wave(1 - islot, 0, 0)
        scatter_wave(islot, 1, 1)
        if has_next:
          fire_wave(1 - islot, 1, 1)

      idx_load(0, 0)
      idx_wait(0)
      fire_wave(0, 0, 0)
      fire_wave(0, 1, 1)

      @pl.loop(0, n_pairs - 2, step=2)
      def _(p):
        do_pair(p, 0, True)
        do_pair(p + 1, 1, True)

      do_pair(n_pairs - 2, 0, True)
      do_pair(n_pairs - 1, 1, False)

    @pl.when(cid == 0)
    def _():
      process(xa_hbm)

    @pl.when(cid == 1)
    def _():
      process(xb_hbm)

    plsc.subcore_barrier()

    wbase = sid * rows_per_sub
    pltpu.sync_copy(acc_sh.at[pl.ds(wbase, rows_per_sub)],
                    agg_hbm.at[pl.ds(cid * two_n + wbase, rows_per_sub)])

  return pl.kernel(body, out_type=out_type, mesh=_mesh(),
                   scratch_types=scratch, compiler_params=_SC_PARAMS)


def _sc_cnt_builder(npad, e_pad):
  two_n = 2 * npad
  rows_per_sub = two_n // NS
  assert rows_per_sub % CHUNK == 0 and rows_per_sub % 8 == 0
  cps = e_pad // (NC * NS * CHUNK)
  n_blocks = cps // PAIR
  assert cps % PAIR == 0 and n_blocks >= 4 and n_blocks % 2 == 0

  out_type = jax.ShapeDtypeStruct((NC * two_n, LANES), jnp.float32)

  scratch = [
      pltpu.VMEM((2, PAIR, CHUNK), jnp.int32),
      pltpu.VMEM((CHUNK, LANES), jnp.float32),
      pltpu.VMEM((CHUNK, LANES), jnp.float32),
      pltpu.VMEM_SHARED((two_n, LANES), jnp.float32),
      pltpu.SemaphoreType.DMA,
      pltpu.SemaphoreType.DMA,
      pltpu.SemaphoreType.DMA,
  ]

  def body(idx_hbm, cnt_hbm, *refs):
    idx_v, ones_v, zero_v, cnt_sh, sem_i0, sem_i1, sem_c = refs
    isems = (sem_i0, sem_i1)

    cid = lax.axis_index("c")
    sid = lax.axis_index("s")

    _zero_fill(zero_v, LANES)

    @pl.loop(0, CHUNK)
    def _(i):
      ones_v.at[pl.ds(i, 1), pl.ds(0, LANES)][...] = jnp.ones(
          (1, LANES), jnp.float32)

    zbase = sid * rows_per_sub

    @pl.loop(0, rows_per_sub, step=CHUNK)
    def _(k):
      pltpu.sync_copy(zero_v, cnt_sh.at[pl.ds(zbase + k, CHUNK)])

    plsc.subcore_barrier()

    row0 = (cid * NS + sid) * cps

    def idx_load(q, slot):
      pltpu.async_copy(idx_hbm.at[pl.ds(row0 + q * PAIR, PAIR)],
                       idx_v.at[slot], isems[slot])

    def idx_wait(slot):
      pltpu.make_async_copy(idx_hbm.at[pl.ds(0, PAIR)], idx_v.at[slot],
                            isems[slot]).wait()

    def count_block(slot):
      for j in range(PAIR):
        pltpu.sync_copy(ones_v, cnt_sh.at[idx_v.at[slot, j]], add=True)

    def drain_block():
      pass

    idx_load(0, 0)
    idx_wait(0)

    @pl.loop(0, n_blocks - 2, step=2)
    def _(q):
      idx_load(q + 1, 1)
      count_block(0)
      idx_wait(1)
      drain_block()
      idx_load(q + 2, 0)
      count_block(1)
      idx_wait(0)
      drain_block()

    idx_load(n_blocks - 1, 1)
    count_block(0)
    idx_wait(1)
    drain_block()
    count_block(1)
    drain_block()

    plsc.subcore_barrier()

    pltpu.sync_copy(cnt_sh.at[pl.ds(zbase, rows_per_sub)],
                    cnt_hbm.at[pl.ds(cid * two_n + zbase, rows_per_sub)])

  return pl.kernel(body, out_type=out_type, mesh=_mesh(),
                   scratch_types=scratch, compiler_params=_SC_PARAMS)


def _tc_layer(x, agg, cnt, w_rel, w_root, b, wc=None, bc=None, blk=1024,
              split_out=False):
  n, d = x.shape
  r_count, _, h = w_rel.shape
  dh = d // 2
  grid = (n // blk,)
  out_dim = wc.shape[1] if wc is not None else h

  in_specs = [pl.BlockSpec((blk, d), lambda i: (i, 0))]
  for c in range(NC):
    for r in range(r_count):
      in_specs.append(pl.BlockSpec(
          (1, blk, dh),
          functools.partial(lambda i, k: (k, i, 0), k=c * r_count + r)))
  for c in range(NC):
    for r in range(r_count):
      in_specs.append(pl.BlockSpec(
          (1, 1, blk, LANES),
          functools.partial(lambda i, c, r: (c, r, i, 0), c=c, r=r)))
  in_specs.append(pl.BlockSpec((r_count, d, h), lambda i: (0, 0, 0)))
  in_specs.append(pl.BlockSpec((d, h), lambda i: (0, 0)))
  in_specs.append(pl.BlockSpec((1, h), lambda i: (0, 0)))
  extra = []
  if wc is not None:
    in_specs.append(pl.BlockSpec((h, out_dim), lambda i: (0, 0)))
    in_specs.append(pl.BlockSpec((1, out_dim), lambda i: (0, 0)))
    extra = [wc, bc.reshape(1, -1)]

  def body(*refs):
    x_ref = refs[0]
    agg_refs = refs[1:1 + NC * r_count]
    cnt_refs = refs[1 + NC * r_count:1 + 2 * NC * r_count]
    base = 1 + 2 * NC * r_count
    w_rel_ref, w_root_ref, b_ref = refs[base], refs[base + 1], refs[base + 2]
    if wc is not None:
      wc_ref, bc_ref = refs[base + 3], refs[base + 4]
    o_ref = refs[-1]

    acc = jnp.dot(x_ref[...], w_root_ref[...],
                  preferred_element_type=jnp.float32,
                  precision=lax.Precision.HIGHEST) + b_ref[...]
    for r in range(r_count):
      a = jnp.concatenate(
          [agg_refs[r][0], agg_refs[r_count + r][0]], axis=1)
      total = cnt_refs[r][0, 0, :, 0:1] + cnt_refs[r_count + r][0, 0, :, 0:1]
      inv = 1.0 / jnp.maximum(total, 1.0)
      acc = acc + jnp.dot(a * inv, w_rel_ref[r],
                          preferred_element_type=jnp.float32,
                          precision=lax.Precision.HIGHEST)
    acc = jnp.maximum(acc, 0.0)
    if wc is not None:
      acc = jnp.dot(acc, wc_ref[...],
                    preferred_element_type=jnp.float32,
                    precision=lax.Precision.HIGHEST) + bc_ref[...]
    if split_out:
      o_ref, oa_ref, ob_ref = refs[-3], refs[-2], refs[-1]
      o_ref[...] = acc
      oa_ref[...] = acc[:, :dh]
      ob_ref[...] = acc[:, dh:]
    else:
      o_ref[...] = acc

  args = [x]
  for _ in range(NC * r_count):
    args.append(agg)
  for _ in range(NC * r_count):
    args.append(cnt)
  args += [w_rel, w_root, b.reshape(1, -1)] + extra

  if split_out:
    out_specs = [pl.BlockSpec((blk, out_dim), lambda i: (i, 0)),
                 pl.BlockSpec((blk, dh), lambda i: (i, 0)),
                 pl.BlockSpec((blk, dh), lambda i: (i, 0))]
    out_shape = [jax.ShapeDtypeStruct((n, out_dim), jnp.float32),
                 jax.ShapeDtypeStruct((n, dh), jnp.float32),
                 jax.ShapeDtypeStruct((n, dh), jnp.float32)]
  else:
    out_specs = pl.BlockSpec((blk, out_dim), lambda i: (i, 0))
    out_shape = jax.ShapeDtypeStruct((n, out_dim), jnp.float32)

  return pl.pallas_call(
      body,
      grid=grid,
      in_specs=in_specs,
      out_specs=out_specs,
      out_shape=out_shape,
  )(*args)


def kernel(x, edge_index, edge_attr, W_rel1, W_root1, b1, W_rel2, W_root2, b2,
           Wc, bc):
  n, d = x.shape
  e = edge_index.shape[1]
  dh = d // 2
  r_count = W_rel1.shape[0]

  npad = ((n + 1 + NS * CHUNK - 1) // (NS * CHUNK)) * (NS * CHUNK)
  edges_per_pass = NC * NS * 2 * PAIR * CHUNK
  e_pad = ((e + edges_per_pass - 1) // edges_per_pass) * edges_per_pass

  src = edge_index[0]
  dst = edge_index[1]
  rel = edge_attr[:, -1].astype(jnp.int32)
  idx = rel * npad + dst
  pad = e_pad - e
  src_p = jnp.concatenate([src, jnp.zeros((pad,), jnp.int32)])
  idx_p = jnp.concatenate([idx, jnp.full((pad,), n, jnp.int32)])
  src2 = src_p.reshape(-1, CHUNK)
  idx2 = idx_p.reshape(-1, CHUNK)

  agg_fn = _sc_agg_builder(npad, e_pad, dh)
  cnt_fn = _sc_cnt_builder(npad, e_pad)

  def shape_agg(a):
    return a.reshape(NC * r_count, npad, dh)

  x_p = jnp.zeros((npad, d), jnp.float32).at[:n, :].set(x)
  cnt = cnt_fn(idx2).reshape(NC, r_count, npad, LANES)
  agg1 = agg_fn(x_p[:, :dh], x_p[:, dh:], src2, idx2)
  h, ha, hb = _tc_layer(x_p, shape_agg(agg1), cnt, W_rel1, W_root1, b1,
                        split_out=True)
  agg2 = agg_fn(ha, hb, src2, idx2)
  out = _tc_layer(h, shape_agg(agg2), cnt, W_rel2, W_root2, b2, Wc, bc)
  return out[:n]
